# Initial kernel scaffold; baseline (speedup 1.0000x reference)
import jax
import jax.numpy as jnp
from jax.experimental import pallas as pl


def kernel(x, edge_index, params):
    raise NotImplementedError("write your pallas kernel here")


# speedup vs baseline: 15.9449x; 15.9449x over previous
import functools

import jax
import jax.numpy as jnp
from jax import lax
from jax.experimental import pallas as pl
from jax.experimental.pallas import tpu as pltpu
from jax.experimental.pallas import tpu_sc as plsc

N = 10000
NP = 10240
E = 160000
EP = 163840
D_FEAT = 128
HID = 64
HEADS = 8
FD = 512
FDX = 640
ROWS = 320
GRID = NP // ROWS
W_CH = 40
CH = 128
CHA = 64
CHB = 32
STRIPE = NP // 16


def _ln(x, g, b):
    mu = jnp.mean(x, axis=-1, keepdims=True)
    var = jnp.mean((x - mu) ** 2, axis=-1, keepdims=True)
    return (x - mu) * jax.lax.rsqrt(var + 1e-5) * g + b


def _ln_plain(x):
    mu = jnp.mean(x, axis=-1, keepdims=True)
    var = jnp.mean((x - mu) ** 2, axis=-1, keepdims=True)
    return (x - mu) * jax.lax.rsqrt(var + 1e-5)


def _enc_body(x_ref, w_ref, b_ref, g_ref, be_ref, o_ref):
    h = jnp.dot(x_ref[...], w_ref[...], preferred_element_type=jnp.float32)
    o_ref[...] = jax.nn.relu(_ln(h + b_ref[...], g_ref[...], be_ref[...]))


def _encoder(xp, p):
    return pl.pallas_call(
        _enc_body,
        grid=(GRID,),
        in_specs=[
            pl.BlockSpec((ROWS, D_FEAT), lambda i: (i, 0)),
            pl.BlockSpec((D_FEAT, HID), lambda i: (0, 0)),
            pl.BlockSpec((1, HID), lambda i: (0, 0)),
            pl.BlockSpec((1, HID), lambda i: (0, 0)),
            pl.BlockSpec((1, HID), lambda i: (0, 0)),
        ],
        out_specs=pl.BlockSpec((ROWS, HID), lambda i: (i, 0)),
        out_shape=jax.ShapeDtypeStruct((NP, HID), jnp.float32),
    )(xp, p['enc_W'], p['enc_b'].reshape(1, HID), p['enc_g'].reshape(1, HID),
      p['enc_be'].reshape(1, HID))


def _frac_body(h_ref, w0, b0, g0, e0, w1, b1, g1, e1, w2, b2, g2, e2, ab,
               hhx_ref, s_ref):
    t0 = jax.nn.relu(_ln(jnp.dot(h_ref[...], w0[...],
                                 preferred_element_type=jnp.float32) + b0[...],
                         g0[...], e0[...]))
    t1 = jax.nn.relu(_ln(jnp.dot(t0, w1[...],
                                 preferred_element_type=jnp.float32) + b1[...],
                         g1[...], e1[...]))
    t2 = jax.nn.relu(_ln(jnp.dot(t1, w2[...],
                                 preferred_element_type=jnp.float32) + b2[...],
                         g2[...], e2[...]))
    hh = (t0 + t1 + t2) * (1.0 / 3.0)
    s = jnp.dot(hh, ab[...], preferred_element_type=jnp.float32)
    s_ref[...] = s
    hhx_ref[...] = jnp.concatenate(
        [hh, s[:, 16:32], jnp.zeros((hh.shape[0], FDX - FD - 16),
                                    jnp.float32)], axis=1)


def _fractal(h, lay, ablk):
    t = lay['t']
    full = lambda shape: pl.BlockSpec(shape, lambda i: (0, 0))
    args = [h]
    specs = [pl.BlockSpec((ROWS, HID), lambda i: (i, 0))]
    dims = [HID, FD, FD]
    for d in range(3):
        args += [t[d]['W'], t[d]['b'].reshape(1, FD), t[d]['g'].reshape(1, FD),
                 t[d]['be'].reshape(1, FD)]
        specs += [full((dims[d], FD)), full((1, FD)), full((1, FD)),
                  full((1, FD))]
    args.append(ablk)
    specs.append(full((FD, 128)))
    return pl.pallas_call(
        _frac_body,
        grid=(GRID,),
        in_specs=specs,
        out_specs=[pl.BlockSpec((ROWS, FDX), lambda i: (i, 0)),
                   pl.BlockSpec((ROWS, 128), lambda i: (i, 0))],
        out_shape=[jax.ShapeDtypeStruct((NP, FDX), jnp.float32),
                   jax.ShapeDtypeStruct((NP, 128), jnp.float32)],
    )(*args)


def _t2_body(dp_ref, s_ref, o_ref):
    dinv = 1.0 / (dp_ref[0, :, 0:16] + dp_ref[1, :, 0:16] + 1e-16)
    o_ref[...] = jnp.concatenate(
        [s_ref[:, 0:16], dinv,
         jnp.zeros((dinv.shape[0], 96), jnp.float32)], axis=1)


def _make_t2(denp, s):
    return pl.pallas_call(
        _t2_body,
        grid=(GRID,),
        in_specs=[pl.BlockSpec((2, ROWS, 128), lambda i: (0, i, 0)),
                  pl.BlockSpec((ROWS, 128), lambda i: (i, 0))],
        out_specs=pl.BlockSpec((ROWS, 128), lambda i: (i, 0)),
        out_shape=jax.ShapeDtypeStruct((NP, 128), jnp.float32),
    )(denp, s)


def _post_body(a_ref, hp_ref, r_ref, w1, b1, w2, b2, hl_ref, ro_ref):
    agg = a_ref[0, :, 0:HID] + a_ref[1, :, 0:HID]
    e1 = jax.nn.relu(jnp.dot(agg, w1[...],
                             preferred_element_type=jnp.float32) + b1[...])
    evo = jnp.tanh(jnp.dot(e1, w2[...],
                           preferred_element_type=jnp.float32) + b2[...])
    gate = jax.nn.sigmoid(evo)
    hn = gate * agg + (1.0 - gate) * hp_ref[...]
    ro_ref[...] = r_ref[...] + hn
    hl_ref[...] = _ln_plain(hn)


def _post(accp, h_prev, res, p):
    full = lambda shape: pl.BlockSpec(shape, lambda i: tuple(0 for _ in shape))
    return pl.pallas_call(
        _post_body,
        grid=(GRID,),
        in_specs=[
            pl.BlockSpec((2, ROWS, 128), lambda i: (0, i, 0)),
            pl.BlockSpec((ROWS, HID), lambda i: (i, 0)),
            pl.BlockSpec((ROWS, HID), lambda i: (i, 0)),
            full((HID, 2 * HID)), full((1, 2 * HID)),
            full((2 * HID, HID)), full((1, HID)),
        ],
        out_specs=[pl.BlockSpec((ROWS, HID), lambda i: (i, 0)),
                   pl.BlockSpec((ROWS, HID), lambda i: (i, 0))],
        out_shape=[jax.ShapeDtypeStruct((NP, HID), jnp.float32),
                   jax.ShapeDtypeStruct((NP, HID), jnp.float32)],
    )(accp, h_prev, res, p['evo_W1'], p['evo_b1'].reshape(1, 2 * HID),
      p['evo_W2'], p['evo_b2'].reshape(1, HID))


def _dec_body(r_ref, w_ref, b_ref, o_ref, hm_ref):
    hm = r_ref[...] * (1.0 / 3.0)
    hm_ref[...] = hm
    o_ref[...] = jnp.dot(hm, w_ref[...],
                         preferred_element_type=jnp.float32) + b_ref[...]


def _decode(res, wcat, bcat):
    return pl.pallas_call(
        _dec_body,
        grid=(GRID,),
        in_specs=[
            pl.BlockSpec((ROWS, HID), lambda i: (i, 0)),
            pl.BlockSpec((HID, 128), lambda i: (0, 0)),
            pl.BlockSpec((1, 128), lambda i: (0, 0)),
        ],
        out_specs=[pl.BlockSpec((ROWS, 128), lambda i: (i, 0)),
                   pl.BlockSpec((ROWS, HID), lambda i: (i, 0))],
        out_shape=[jax.ShapeDtypeStruct((NP, 128), jnp.float32),
                   jax.ShapeDtypeStruct((NP, HID), jnp.float32)],
    )(res, wcat, bcat)


_MESH = plsc.VectorSubcoreMesh(core_axis_name="c", subcore_axis_name="s")


def _sc_denom_kernel(s128, srcm, dstm, z128):
    @functools.partial(
        pl.kernel,
        out_type=jax.ShapeDtypeStruct((2, NP, 128), jnp.float32),
        mesh=_MESH,
        scratch_types=[
            pltpu.VMEM((1, CH), jnp.int32),
            pltpu.VMEM((1, CH), jnp.int32),
            pltpu.VMEM((CHA, 128), jnp.float32),
            pltpu.VMEM((CHA, 128), jnp.float32),
            pltpu.VMEM((CH, 128), jnp.float32),
            pltpu.VMEM_SHARED((NP, 128), jnp.float32),
            pltpu.SemaphoreType.DMA,
            pltpu.SemaphoreType.DMA,
        ],
    )
    def k(s_hbm, src_hbm, dst_hbm, z_hbm, out_hbm,
          isrc, idst, rs, rd, ex, dsh, sa, sb):
        cid = lax.axis_index("c")
        sid = lax.axis_index("s")
        wid = cid * 16 + sid

        def zex(e, c):
            for kk in range(8):
                ex[e, pl.ds(kk * 16, 16)] = jnp.zeros((16,), jnp.float32)
            return c

        lax.fori_loop(0, CH, zex, 0)
        pltpu.sync_copy(z_hbm, dsh.at[pl.ds(sid * STRIPE, STRIPE)])
        plsc.subcore_barrier()

        def chunk(j, carry):
            pltpu.sync_copy(src_hbm.at[wid, pl.ds(j, 1)], isrc)
            pltpu.sync_copy(dst_hbm.at[wid, pl.ds(j, 1)], idst)
            for sub in range(CH // CHA):
                ca = pltpu.async_copy(
                    s_hbm.at[isrc.at[0, pl.ds(sub * CHA, CHA)]], rs, sa)
                cb = pltpu.async_copy(
                    s_hbm.at[idst.at[0, pl.ds(sub * CHA, CHA)]], rd, sb)
                ca.wait()
                cb.wait()

                def edge(e, c, _sub=sub):
                    a = rd[e, pl.ds(0, 16)] + rs[e, pl.ds(16, 16)]
                    a = jnp.maximum(a, 0.2 * a)
                    ex[_sub * CHA + e, pl.ds(0, 16)] = jnp.exp(a)
                    return c

                lax.fori_loop(0, CHA, edge, 0)
            pltpu.sync_copy(ex, dsh.at[idst.at[0]], add=True)
            return carry

        lax.fori_loop(0, W_CH, chunk, 0)
        plsc.subcore_barrier()
        pltpu.sync_copy(dsh.at[pl.ds(sid * STRIPE, STRIPE)],
                        out_hbm.at[cid, pl.ds(sid * STRIPE, STRIPE)])

    return k(s128, srcm, dstm, z128)


def _sc_agg_kernel(hhx, t2, srcm, dstm, z128):
    @functools.partial(
        pl.kernel,
        out_type=jax.ShapeDtypeStruct((2, NP, 128), jnp.float32),
        mesh=_MESH,
        scratch_types=[
            pltpu.VMEM((1, CH), jnp.int32),
            pltpu.VMEM((1, CH), jnp.int32),
            pltpu.VMEM((CHB, 128), jnp.float32),
            pltpu.VMEM((CHB, FDX), jnp.float32),
            pltpu.VMEM((CH, 128), jnp.float32),
            pltpu.VMEM_SHARED((NP, 128), jnp.float32),
            pltpu.SemaphoreType.DMA,
            pltpu.SemaphoreType.DMA,
        ],
    )
    def k(hh_hbm, t2_hbm, src_hbm, dst_hbm, z_hbm, out_hbm,
          isrc, idst, rd, rows, msg, ash, sa, sb):
        cid = lax.axis_index("c")
        sid = lax.axis_index("s")
        wid = cid * 16 + sid

        def zmsg(e, c):
            for kk in range(8):
                msg[e, pl.ds(kk * 16, 16)] = jnp.zeros((16,), jnp.float32)
            return c

        lax.fori_loop(0, CH, zmsg, 0)
        pltpu.sync_copy(z_hbm, ash.at[pl.ds(sid * STRIPE, STRIPE)])
        plsc.subcore_barrier()

        def chunk(j, carry):
            pltpu.sync_copy(src_hbm.at[wid, pl.ds(j, 1)], isrc)
            pltpu.sync_copy(dst_hbm.at[wid, pl.ds(j, 1)], idst)
            for sub in range(CH // CHB):
                cr = pltpu.async_copy(
                    hh_hbm.at[isrc.at[0, pl.ds(sub * CHB, CHB)]], rows, sa)
                cb = pltpu.async_copy(
                    t2_hbm.at[idst.at[0, pl.ds(sub * CHB, CHB)]], rd, sb)
                cr.wait()
                cb.wait()

                def edge(e, c, _sub=sub):
                    a = rd[e, pl.ds(0, 16)] + rows[e, pl.ds(FD, 16)]
                    a = jnp.maximum(a, 0.2 * a)
                    w = jnp.exp(a) * rd[e, pl.ds(16, 16)] * 0.125
                    acc0 = jnp.zeros((16,), jnp.float32)
                    acc1 = jnp.zeros((16,), jnp.float32)
                    acc2 = jnp.zeros((16,), jnp.float32)
                    acc3 = jnp.zeros((16,), jnp.float32)
                    for h in range(HEADS):
                        wb = lax.gather(
                            w, jnp.full((16, 1), h, jnp.int32),
                            lax.GatherDimensionNumbers(
                                offset_dims=(), collapsed_slice_dims=(0,),
                                start_index_map=(0,)),
                            (1,),
                            mode=lax.GatherScatterMode.PROMISE_IN_BOUNDS)
                        acc0 = acc0 + wb * rows[e, pl.ds(h * HID, 16)]
                        acc1 = acc1 + wb * rows[e, pl.ds(h * HID + 16, 16)]
                        acc2 = acc2 + wb * rows[e, pl.ds(h * HID + 32, 16)]
                        acc3 = acc3 + wb * rows[e, pl.ds(h * HID + 48, 16)]
                    base = _sub * CHB + e
                    msg[base, pl.ds(0, 16)] = acc0
                    msg[base, pl.ds(16, 16)] = acc1
                    msg[base, pl.ds(32, 16)] = acc2
                    msg[base, pl.ds(48, 16)] = acc3
                    return c

                lax.fori_loop(0, CHB, edge, 0)
            pltpu.sync_copy(msg, ash.at[idst.at[0]], add=True)
            return carry

        lax.fori_loop(0, W_CH, chunk, 0)
        plsc.subcore_barrier()
        pltpu.sync_copy(ash.at[pl.ds(sid * STRIPE, STRIPE)],
                        out_hbm.at[cid, pl.ds(sid * STRIPE, STRIPE)])

    return k(hhx, t2, srcm, dstm, z128)


def _make_ablk(att):
    att_i = att[0, :, :HID]
    att_j = att[0, :, HID:]
    eye = jnp.eye(HEADS, dtype=jnp.float32)
    bi = (att_i[:, :, None] * eye[:, None, :]).reshape(FD, HEADS)
    bj = (att_j[:, :, None] * eye[:, None, :]).reshape(FD, HEADS)
    return jnp.concatenate([bi, bi, bj, bj,
                            jnp.zeros((FD, 96), jnp.float32)], axis=1)


def kernel(x, edge_index, params):
    p = params
    xp = jnp.pad(x, ((0, NP - N), (0, 0)))
    srcf = jnp.concatenate(
        [edge_index[0].astype(jnp.int32), jnp.full((EP - E,), N, jnp.int32)])
    dstf = jnp.concatenate(
        [edge_index[1].astype(jnp.int32), jnp.full((EP - E,), N, jnp.int32)])
    src_m = srcf.reshape(32, W_CH, CH)
    dst_m = dstf.reshape(32, W_CH, CH)
    z128 = jnp.zeros((STRIPE, 128), jnp.float32)

    h = _encoder(xp, p)
    res = jnp.zeros((NP, HID), jnp.float32)
    for lay in p['layers']:
        ablk = _make_ablk(lay['att'])
        hhx, s128 = _fractal(h, lay, ablk)
        denp = _sc_denom_kernel(s128, src_m, dst_m, z128)
        t2 = _make_t2(denp, s128)
        accp = _sc_agg_kernel(hhx, t2, src_m, dst_m, z128)
        h, res = _post(accp, h, res, p)

    wcat = jnp.concatenate(
        [p['dec_state_W'], p['dec_importance_W'], p['dec_chaos_W'],
         p['dec_evolution_W'],
         jnp.zeros((HID, 128 - 3 - HID), jnp.float32)], axis=1)
    bcat = jnp.concatenate(
        [p['dec_state_b'], p['dec_importance_b'], p['dec_chaos_b'],
         p['dec_evolution_b'],
         jnp.zeros((128 - 3 - HID,), jnp.float32)]).reshape(1, 128)
    out128, hm = _decode(res, wcat, bcat)
    return (out128[:N, 0:1], out128[:N, 1:2], out128[:N, 2:3],
            out128[:N, 3:3 + HID], hm[:N])


# speedup vs baseline: 17.7953x; 1.1160x over previous
import functools

import jax
import jax.numpy as jnp
from jax import lax
from jax.experimental import pallas as pl
from jax.experimental.pallas import tpu as pltpu
from jax.experimental.pallas import tpu_sc as plsc

N = 10000
NP = 10240
E = 160000
EP = 163840
D_FEAT = 128
HID = 64
HEADS = 8
FD = 512
FDX = 640
ROWS = 320
GRID = NP // ROWS
W_CH = 40
CH = 128
CHA = 64
CHB = 16
STRIPE = NP // 16


def _ln(x, g, b):
    mu = jnp.mean(x, axis=-1, keepdims=True)
    var = jnp.mean((x - mu) ** 2, axis=-1, keepdims=True)
    return (x - mu) * jax.lax.rsqrt(var + 1e-5) * g + b


def _ln_plain(x):
    mu = jnp.mean(x, axis=-1, keepdims=True)
    var = jnp.mean((x - mu) ** 2, axis=-1, keepdims=True)
    return (x - mu) * jax.lax.rsqrt(var + 1e-5)


def _enc_body(x_ref, w_ref, b_ref, g_ref, be_ref, o_ref):
    h = jnp.dot(x_ref[...], w_ref[...], preferred_element_type=jnp.float32)
    o_ref[...] = jax.nn.relu(_ln(h + b_ref[...], g_ref[...], be_ref[...]))


def _encoder(xp, p):
    return pl.pallas_call(
        _enc_body,
        grid=(GRID,),
        in_specs=[
            pl.BlockSpec((ROWS, D_FEAT), lambda i: (i, 0)),
            pl.BlockSpec((D_FEAT, HID), lambda i: (0, 0)),
            pl.BlockSpec((1, HID), lambda i: (0, 0)),
            pl.BlockSpec((1, HID), lambda i: (0, 0)),
            pl.BlockSpec((1, HID), lambda i: (0, 0)),
        ],
        out_specs=pl.BlockSpec((ROWS, HID), lambda i: (i, 0)),
        out_shape=jax.ShapeDtypeStruct((NP, HID), jnp.float32),
    )(xp, p['enc_W'], p['enc_b'].reshape(1, HID), p['enc_g'].reshape(1, HID),
      p['enc_be'].reshape(1, HID))


def _frac_body(h_ref, w0, b0, g0, e0, w1, b1, g1, e1, w2, b2, g2, e2, ab,
               hhx_ref, s_ref):
    t0 = jax.nn.relu(_ln(jnp.dot(h_ref[...], w0[...],
                                 preferred_element_type=jnp.float32) + b0[...],
                         g0[...], e0[...]))
    t1 = jax.nn.relu(_ln(jnp.dot(t0, w1[...],
                                 preferred_element_type=jnp.float32) + b1[...],
                         g1[...], e1[...]))
    t2 = jax.nn.relu(_ln(jnp.dot(t1, w2[...],
                                 preferred_element_type=jnp.float32) + b2[...],
                         g2[...], e2[...]))
    hh = (t0 + t1 + t2) * (1.0 / 3.0)
    s = jnp.dot(hh, ab[...], preferred_element_type=jnp.float32)
    s_ref[...] = s
    hhx_ref[...] = jnp.concatenate(
        [hh, s[:, 16:32], jnp.zeros((hh.shape[0], FDX - FD - 16),
                                    jnp.float32)], axis=1)


def _fractal(h, lay, ablk):
    t = lay['t']
    full = lambda shape: pl.BlockSpec(shape, lambda i: (0, 0))
    args = [h]
    specs = [pl.BlockSpec((ROWS, HID), lambda i: (i, 0))]
    dims = [HID, FD, FD]
    for d in range(3):
        args += [t[d]['W'], t[d]['b'].reshape(1, FD), t[d]['g'].reshape(1, FD),
                 t[d]['be'].reshape(1, FD)]
        specs += [full((dims[d], FD)), full((1, FD)), full((1, FD)),
                  full((1, FD))]
    args.append(ablk)
    specs.append(full((FD, 128)))
    return pl.pallas_call(
        _frac_body,
        grid=(GRID,),
        in_specs=specs,
        out_specs=[pl.BlockSpec((ROWS, FDX), lambda i: (i, 0)),
                   pl.BlockSpec((ROWS, 128), lambda i: (i, 0))],
        out_shape=[jax.ShapeDtypeStruct((NP, FDX), jnp.float32),
                   jax.ShapeDtypeStruct((NP, 128), jnp.float32)],
    )(*args)


def _t2_body(dp_ref, s_ref, o_ref):
    dinv = 1.0 / (dp_ref[0, :, 0:16] + dp_ref[1, :, 0:16] + 1e-16)
    o_ref[...] = jnp.concatenate(
        [s_ref[:, 0:16], dinv,
         jnp.zeros((dinv.shape[0], 96), jnp.float32)], axis=1)


def _make_t2(denp, s):
    return pl.pallas_call(
        _t2_body,
        grid=(GRID,),
        in_specs=[pl.BlockSpec((2, ROWS, 128), lambda i: (0, i, 0)),
                  pl.BlockSpec((ROWS, 128), lambda i: (i, 0))],
        out_specs=pl.BlockSpec((ROWS, 128), lambda i: (i, 0)),
        out_shape=jax.ShapeDtypeStruct((NP, 128), jnp.float32),
    )(denp, s)


def _post_body(a_ref, hp_ref, r_ref, w1, b1, w2, b2, hl_ref, ro_ref):
    agg = a_ref[0, :, 0:HID] + a_ref[1, :, 0:HID]
    e1 = jax.nn.relu(jnp.dot(agg, w1[...],
                             preferred_element_type=jnp.float32) + b1[...])
    evo = jnp.tanh(jnp.dot(e1, w2[...],
                           preferred_element_type=jnp.float32) + b2[...])
    gate = jax.nn.sigmoid(evo)
    hn = gate * agg + (1.0 - gate) * hp_ref[...]
    ro_ref[...] = r_ref[...] + hn
    hl_ref[...] = _ln_plain(hn)


def _post(accp, h_prev, res, p):
    full = lambda shape: pl.BlockSpec(shape, lambda i: tuple(0 for _ in shape))
    return pl.pallas_call(
        _post_body,
        grid=(GRID,),
        in_specs=[
            pl.BlockSpec((2, ROWS, 128), lambda i: (0, i, 0)),
            pl.BlockSpec((ROWS, HID), lambda i: (i, 0)),
            pl.BlockSpec((ROWS, HID), lambda i: (i, 0)),
            full((HID, 2 * HID)), full((1, 2 * HID)),
            full((2 * HID, HID)), full((1, HID)),
        ],
        out_specs=[pl.BlockSpec((ROWS, HID), lambda i: (i, 0)),
                   pl.BlockSpec((ROWS, HID), lambda i: (i, 0))],
        out_shape=[jax.ShapeDtypeStruct((NP, HID), jnp.float32),
                   jax.ShapeDtypeStruct((NP, HID), jnp.float32)],
    )(accp, h_prev, res, p['evo_W1'], p['evo_b1'].reshape(1, 2 * HID),
      p['evo_W2'], p['evo_b2'].reshape(1, HID))


def _dec_body(r_ref, w_ref, b_ref, o_ref, hm_ref):
    hm = r_ref[...] * (1.0 / 3.0)
    hm_ref[...] = hm
    o_ref[...] = jnp.dot(hm, w_ref[...],
                         preferred_element_type=jnp.float32) + b_ref[...]


def _decode(res, wcat, bcat):
    return pl.pallas_call(
        _dec_body,
        grid=(GRID,),
        in_specs=[
            pl.BlockSpec((ROWS, HID), lambda i: (i, 0)),
            pl.BlockSpec((HID, 128), lambda i: (0, 0)),
            pl.BlockSpec((1, 128), lambda i: (0, 0)),
        ],
        out_specs=[pl.BlockSpec((ROWS, 128), lambda i: (i, 0)),
                   pl.BlockSpec((ROWS, HID), lambda i: (i, 0))],
        out_shape=[jax.ShapeDtypeStruct((NP, 128), jnp.float32),
                   jax.ShapeDtypeStruct((NP, HID), jnp.float32)],
    )(res, wcat, bcat)


_MESH = plsc.VectorSubcoreMesh(core_axis_name="c", subcore_axis_name="s")


def _sc_denom_kernel(s128, srcm, dstm, z128):
    @functools.partial(
        pl.kernel,
        out_type=jax.ShapeDtypeStruct((2, NP, 128), jnp.float32),
        mesh=_MESH,
        scratch_types=[
            pltpu.VMEM((1, CH), jnp.int32),
            pltpu.VMEM((1, CH), jnp.int32),
            pltpu.VMEM((CHA, 128), jnp.float32),
            pltpu.VMEM((CHA, 128), jnp.float32),
            pltpu.VMEM((CH, 128), jnp.float32),
            pltpu.VMEM_SHARED((NP, 128), jnp.float32),
            pltpu.SemaphoreType.DMA,
            pltpu.SemaphoreType.DMA,
        ],
    )
    def k(s_hbm, src_hbm, dst_hbm, z_hbm, out_hbm,
          isrc, idst, rs, rd, ex, dsh, sa, sb):
        cid = lax.axis_index("c")
        sid = lax.axis_index("s")
        wid = cid * 16 + sid

        def zex(e, c):
            for kk in range(8):
                ex[e, pl.ds(kk * 16, 16)] = jnp.zeros((16,), jnp.float32)
            return c

        lax.fori_loop(0, CH, zex, 0)
        pltpu.sync_copy(z_hbm, dsh.at[pl.ds(sid * STRIPE, STRIPE)])
        plsc.subcore_barrier()

        def chunk(j, carry):
            pltpu.sync_copy(src_hbm.at[wid, pl.ds(j, 1)], isrc)
            pltpu.sync_copy(dst_hbm.at[wid, pl.ds(j, 1)], idst)
            for sub in range(CH // CHA):
                ca = pltpu.async_copy(
                    s_hbm.at[isrc.at[0, pl.ds(sub * CHA, CHA)]], rs, sa)
                cb = pltpu.async_copy(
                    s_hbm.at[idst.at[0, pl.ds(sub * CHA, CHA)]], rd, sb)
                ca.wait()
                cb.wait()

                def edge(e, c, _sub=sub):
                    a = rd[e, pl.ds(0, 16)] + rs[e, pl.ds(16, 16)]
                    a = jnp.maximum(a, 0.2 * a)
                    ex[_sub * CHA + e, pl.ds(0, 16)] = jnp.exp(a)
                    return c

                lax.fori_loop(0, CHA, edge, 0)
            pltpu.sync_copy(ex, dsh.at[idst.at[0]], add=True)
            return carry

        lax.fori_loop(0, W_CH, chunk, 0)
        plsc.subcore_barrier()
        pltpu.sync_copy(dsh.at[pl.ds(sid * STRIPE, STRIPE)],
                        out_hbm.at[cid, pl.ds(sid * STRIPE, STRIPE)])

    return k(s128, srcm, dstm, z128)


def _sc_agg_kernel(hhx, t2, srcm, dstm, z128):
    @functools.partial(
        pl.kernel,
        out_type=jax.ShapeDtypeStruct((2, NP, 128), jnp.float32),
        mesh=_MESH,
        scratch_types=[
            pltpu.VMEM((1, CH), jnp.int32),
            pltpu.VMEM((1, CH), jnp.int32),
            pltpu.VMEM((CHB, 128), jnp.float32),
            pltpu.VMEM((CHB, 128), jnp.float32),
            pltpu.VMEM((CHB, FDX), jnp.float32),
            pltpu.VMEM((CHB, FDX), jnp.float32),
            pltpu.VMEM((CH, 128), jnp.float32),
            pltpu.VMEM_SHARED((NP, 128), jnp.float32),
            pltpu.SemaphoreType.DMA,
            pltpu.SemaphoreType.DMA,
            pltpu.SemaphoreType.DMA,
            pltpu.SemaphoreType.DMA,
        ],
    )
    def k(hh_hbm, t2_hbm, src_hbm, dst_hbm, z_hbm, out_hbm,
          isrc, idst, rd0, rd1, rows0, rows1, msg, ash, sa, sb, sc2, sd2):
        cid = lax.axis_index("c")
        sid = lax.axis_index("s")
        wid = cid * 16 + sid

        def zmsg(e, c):
            for kk in range(8):
                msg[e, pl.ds(kk * 16, 16)] = jnp.zeros((16,), jnp.float32)
            return c

        lax.fori_loop(0, CH, zmsg, 0)
        pltpu.sync_copy(z_hbm, ash.at[pl.ds(sid * STRIPE, STRIPE)])
        plsc.subcore_barrier()

        def chunk(j, carry):
            pltpu.sync_copy(src_hbm.at[wid, pl.ds(j, 1)], isrc)
            pltpu.sync_copy(dst_hbm.at[wid, pl.ds(j, 1)], idst)
            nsub = CH // CHB
            rows_b = [rows0, rows1]
            rd_b = [rd0, rd1]
            sr_b = [sa, sc2]
            sd_b = [sb, sd2]
            pend = [None] * nsub
            pend[0] = (
                pltpu.async_copy(hh_hbm.at[isrc.at[0, pl.ds(0, CHB)]],
                                 rows0, sa),
                pltpu.async_copy(t2_hbm.at[idst.at[0, pl.ds(0, CHB)]],
                                 rd0, sb))
            for sub in range(nsub):
                if sub + 1 < nsub:
                    nb = (sub + 1) % 2
                    pend[sub + 1] = (
                        pltpu.async_copy(
                            hh_hbm.at[isrc.at[0, pl.ds((sub + 1) * CHB, CHB)]],
                            rows_b[nb], sr_b[nb]),
                        pltpu.async_copy(
                            t2_hbm.at[idst.at[0, pl.ds((sub + 1) * CHB, CHB)]],
                            rd_b[nb], sd_b[nb]))
                pend[sub][0].wait()
                pend[sub][1].wait()
                rows = rows_b[sub % 2]
                rd = rd_b[sub % 2]

                def edge(e, c, _sub=sub, rows=rows, rd=rd):
                    a = rd[e, pl.ds(0, 16)] + rows[e, pl.ds(FD, 16)]
                    a = jnp.maximum(a, 0.2 * a)
                    w = jnp.exp(a) * rd[e, pl.ds(16, 16)] * 0.125
                    acc0 = jnp.zeros((16,), jnp.float32)
                    acc1 = jnp.zeros((16,), jnp.float32)
                    acc2 = jnp.zeros((16,), jnp.float32)
                    acc3 = jnp.zeros((16,), jnp.float32)
                    for h in range(HEADS):
                        wb = lax.gather(
                            w, jnp.full((16, 1), h, jnp.int32),
                            lax.GatherDimensionNumbers(
                                offset_dims=(), collapsed_slice_dims=(0,),
                                start_index_map=(0,)),
                            (1,),
                            mode=lax.GatherScatterMode.PROMISE_IN_BOUNDS)
                        acc0 = acc0 + wb * rows[e, pl.ds(h * HID, 16)]
                        acc1 = acc1 + wb * rows[e, pl.ds(h * HID + 16, 16)]
                        acc2 = acc2 + wb * rows[e, pl.ds(h * HID + 32, 16)]
                        acc3 = acc3 + wb * rows[e, pl.ds(h * HID + 48, 16)]
                    base = _sub * CHB + e
                    msg[base, pl.ds(0, 16)] = acc0
                    msg[base, pl.ds(16, 16)] = acc1
                    msg[base, pl.ds(32, 16)] = acc2
                    msg[base, pl.ds(48, 16)] = acc3
                    return c

                lax.fori_loop(0, CHB, edge, 0)
            pltpu.sync_copy(msg, ash.at[idst.at[0]], add=True)
            return carry

        lax.fori_loop(0, W_CH, chunk, 0)
        plsc.subcore_barrier()
        pltpu.sync_copy(ash.at[pl.ds(sid * STRIPE, STRIPE)],
                        out_hbm.at[cid, pl.ds(sid * STRIPE, STRIPE)])

    return k(hhx, t2, srcm, dstm, z128)


def _make_ablk(att):
    att_i = att[0, :, :HID]
    att_j = att[0, :, HID:]
    eye = jnp.eye(HEADS, dtype=jnp.float32)
    bi = (att_i[:, :, None] * eye[:, None, :]).reshape(FD, HEADS)
    bj = (att_j[:, :, None] * eye[:, None, :]).reshape(FD, HEADS)
    return jnp.concatenate([bi, bi, bj, bj,
                            jnp.zeros((FD, 96), jnp.float32)], axis=1)


def kernel(x, edge_index, params):
    p = params
    xp = jnp.pad(x, ((0, NP - N), (0, 0)))
    srcf = jnp.concatenate(
        [edge_index[0].astype(jnp.int32), jnp.full((EP - E,), N, jnp.int32)])
    dstf = jnp.concatenate(
        [edge_index[1].astype(jnp.int32), jnp.full((EP - E,), N, jnp.int32)])
    src_m = srcf.reshape(32, W_CH, CH)
    dst_m = dstf.reshape(32, W_CH, CH)
    z128 = jnp.zeros((STRIPE, 128), jnp.float32)

    h = _encoder(xp, p)
    res = jnp.zeros((NP, HID), jnp.float32)
    for lay in p['layers']:
        ablk = _make_ablk(lay['att'])
        hhx, s128 = _fractal(h, lay, ablk)
        denp = _sc_denom_kernel(s128, src_m, dst_m, z128)
        t2 = _make_t2(denp, s128)
        accp = _sc_agg_kernel(hhx, t2, src_m, dst_m, z128)
        h, res = _post(accp, h, res, p)

    wcat = jnp.concatenate(
        [p['dec_state_W'], p['dec_importance_W'], p['dec_chaos_W'],
         p['dec_evolution_W'],
         jnp.zeros((HID, 128 - 3 - HID), jnp.float32)], axis=1)
    bcat = jnp.concatenate(
        [p['dec_state_b'], p['dec_importance_b'], p['dec_chaos_b'],
         p['dec_evolution_b'],
         jnp.zeros((128 - 3 - HID,), jnp.float32)]).reshape(1, 128)
    out128, hm = _decode(res, wcat, bcat)
    return (out128[:N, 0:1], out128[:N, 1:2], out128[:N, 2:3],
            out128[:N, 3:3 + HID], hm[:N])


# speedup vs baseline: 17.9701x; 1.0098x over previous
import functools

import jax
import jax.numpy as jnp
from jax import lax
from jax.experimental import pallas as pl
from jax.experimental.pallas import tpu as pltpu
from jax.experimental.pallas import tpu_sc as plsc

N = 10000
NP = 10240
E = 160000
EP = 163840
D_FEAT = 128
HID = 64
HEADS = 8
FD = 512
FDX = 640
ROWS = 320
GRID = NP // ROWS
W_CH = 40
CH = 128
CHA = 32
CHB = 16
STRIPE = NP // 16


def _ln(x, g, b):
    mu = jnp.mean(x, axis=-1, keepdims=True)
    var = jnp.mean((x - mu) ** 2, axis=-1, keepdims=True)
    return (x - mu) * jax.lax.rsqrt(var + 1e-5) * g + b


def _ln_plain(x):
    mu = jnp.mean(x, axis=-1, keepdims=True)
    var = jnp.mean((x - mu) ** 2, axis=-1, keepdims=True)
    return (x - mu) * jax.lax.rsqrt(var + 1e-5)


def _enc_body(x_ref, w_ref, b_ref, g_ref, be_ref, o_ref):
    h = jnp.dot(x_ref[...], w_ref[...], preferred_element_type=jnp.float32)
    o_ref[...] = jax.nn.relu(_ln(h + b_ref[...], g_ref[...], be_ref[...]))


def _encoder(xp, p):
    return pl.pallas_call(
        _enc_body,
        grid=(GRID,),
        in_specs=[
            pl.BlockSpec((ROWS, D_FEAT), lambda i: (i, 0)),
            pl.BlockSpec((D_FEAT, HID), lambda i: (0, 0)),
            pl.BlockSpec((1, HID), lambda i: (0, 0)),
            pl.BlockSpec((1, HID), lambda i: (0, 0)),
            pl.BlockSpec((1, HID), lambda i: (0, 0)),
        ],
        out_specs=pl.BlockSpec((ROWS, HID), lambda i: (i, 0)),
        out_shape=jax.ShapeDtypeStruct((NP, HID), jnp.float32),
    )(xp, p['enc_W'], p['enc_b'].reshape(1, HID), p['enc_g'].reshape(1, HID),
      p['enc_be'].reshape(1, HID))


def _frac_body(h_ref, w0, b0, g0, e0, w1, b1, g1, e1, w2, b2, g2, e2, ab,
               hhx_ref, s_ref):
    t0 = jax.nn.relu(_ln(jnp.dot(h_ref[...], w0[...],
                                 preferred_element_type=jnp.float32) + b0[...],
                         g0[...], e0[...]))
    t1 = jax.nn.relu(_ln(jnp.dot(t0, w1[...],
                                 preferred_element_type=jnp.float32) + b1[...],
                         g1[...], e1[...]))
    t2 = jax.nn.relu(_ln(jnp.dot(t1, w2[...],
                                 preferred_element_type=jnp.float32) + b2[...],
                         g2[...], e2[...]))
    hh = (t0 + t1 + t2) * (1.0 / 3.0)
    s = jnp.dot(hh, ab[...], preferred_element_type=jnp.float32)
    s_ref[...] = s
    hhx_ref[...] = jnp.concatenate(
        [hh, s[:, 16:32], jnp.zeros((hh.shape[0], FDX - FD - 16),
                                    jnp.float32)], axis=1)


def _fractal(h, lay, ablk):
    t = lay['t']
    full = lambda shape: pl.BlockSpec(shape, lambda i: (0, 0))
    args = [h]
    specs = [pl.BlockSpec((ROWS, HID), lambda i: (i, 0))]
    dims = [HID, FD, FD]
    for d in range(3):
        args += [t[d]['W'], t[d]['b'].reshape(1, FD), t[d]['g'].reshape(1, FD),
                 t[d]['be'].reshape(1, FD)]
        specs += [full((dims[d], FD)), full((1, FD)), full((1, FD)),
                  full((1, FD))]
    args.append(ablk)
    specs.append(full((FD, 128)))
    return pl.pallas_call(
        _frac_body,
        grid=(GRID,),
        in_specs=specs,
        out_specs=[pl.BlockSpec((ROWS, FDX), lambda i: (i, 0)),
                   pl.BlockSpec((ROWS, 128), lambda i: (i, 0))],
        out_shape=[jax.ShapeDtypeStruct((NP, FDX), jnp.float32),
                   jax.ShapeDtypeStruct((NP, 128), jnp.float32)],
    )(*args)


def _t2_body(dp_ref, s_ref, o_ref):
    dinv = 1.0 / (dp_ref[0, :, 0:16] + dp_ref[1, :, 0:16] + 1e-16)
    o_ref[...] = jnp.concatenate(
        [s_ref[:, 0:16], dinv,
         jnp.zeros((dinv.shape[0], 96), jnp.float32)], axis=1)


def _make_t2(denp, s):
    return pl.pallas_call(
        _t2_body,
        grid=(GRID,),
        in_specs=[pl.BlockSpec((2, ROWS, 128), lambda i: (0, i, 0)),
                  pl.BlockSpec((ROWS, 128), lambda i: (i, 0))],
        out_specs=pl.BlockSpec((ROWS, 128), lambda i: (i, 0)),
        out_shape=jax.ShapeDtypeStruct((NP, 128), jnp.float32),
    )(denp, s)


def _post_body(a_ref, hp_ref, r_ref, w1, b1, w2, b2, hl_ref, ro_ref):
    agg = a_ref[0, :, 0:HID] + a_ref[1, :, 0:HID]
    e1 = jax.nn.relu(jnp.dot(agg, w1[...],
                             preferred_element_type=jnp.float32) + b1[...])
    evo = jnp.tanh(jnp.dot(e1, w2[...],
                           preferred_element_type=jnp.float32) + b2[...])
    gate = jax.nn.sigmoid(evo)
    hn = gate * agg + (1.0 - gate) * hp_ref[...]
    ro_ref[...] = r_ref[...] + hn
    hl_ref[...] = _ln_plain(hn)


def _post(accp, h_prev, res, p):
    full = lambda shape: pl.BlockSpec(shape, lambda i: tuple(0 for _ in shape))
    return pl.pallas_call(
        _post_body,
        grid=(GRID,),
        in_specs=[
            pl.BlockSpec((2, ROWS, 128), lambda i: (0, i, 0)),
            pl.BlockSpec((ROWS, HID), lambda i: (i, 0)),
            pl.BlockSpec((ROWS, HID), lambda i: (i, 0)),
            full((HID, 2 * HID)), full((1, 2 * HID)),
            full((2 * HID, HID)), full((1, HID)),
        ],
        out_specs=[pl.BlockSpec((ROWS, HID), lambda i: (i, 0)),
                   pl.BlockSpec((ROWS, HID), lambda i: (i, 0))],
        out_shape=[jax.ShapeDtypeStruct((NP, HID), jnp.float32),
                   jax.ShapeDtypeStruct((NP, HID), jnp.float32)],
    )(accp, h_prev, res, p['evo_W1'], p['evo_b1'].reshape(1, 2 * HID),
      p['evo_W2'], p['evo_b2'].reshape(1, HID))


def _dec_body(r_ref, w_ref, b_ref, o_ref, hm_ref):
    hm = r_ref[...] * (1.0 / 3.0)
    hm_ref[...] = hm
    o_ref[...] = jnp.dot(hm, w_ref[...],
                         preferred_element_type=jnp.float32) + b_ref[...]


def _decode(res, wcat, bcat):
    return pl.pallas_call(
        _dec_body,
        grid=(GRID,),
        in_specs=[
            pl.BlockSpec((ROWS, HID), lambda i: (i, 0)),
            pl.BlockSpec((HID, 128), lambda i: (0, 0)),
            pl.BlockSpec((1, 128), lambda i: (0, 0)),
        ],
        out_specs=[pl.BlockSpec((ROWS, 128), lambda i: (i, 0)),
                   pl.BlockSpec((ROWS, HID), lambda i: (i, 0))],
        out_shape=[jax.ShapeDtypeStruct((NP, 128), jnp.float32),
                   jax.ShapeDtypeStruct((NP, HID), jnp.float32)],
    )(res, wcat, bcat)


_MESH = plsc.VectorSubcoreMesh(core_axis_name="c", subcore_axis_name="s")


def _sc_denom_kernel(s128, srcm, dstm, z128):
    @functools.partial(
        pl.kernel,
        out_type=jax.ShapeDtypeStruct((2, NP, 128), jnp.float32),
        mesh=_MESH,
        scratch_types=[
            pltpu.VMEM((1, CH), jnp.int32),
            pltpu.VMEM((1, CH), jnp.int32),
            pltpu.VMEM((CHA, 128), jnp.float32),
            pltpu.VMEM((CHA, 128), jnp.float32),
            pltpu.VMEM((CHA, 128), jnp.float32),
            pltpu.VMEM((CHA, 128), jnp.float32),
            pltpu.VMEM((CH, 128), jnp.float32),
            pltpu.VMEM_SHARED((NP, 128), jnp.float32),
            pltpu.SemaphoreType.DMA,
            pltpu.SemaphoreType.DMA,
            pltpu.SemaphoreType.DMA,
            pltpu.SemaphoreType.DMA,
        ],
    )
    def k(s_hbm, src_hbm, dst_hbm, z_hbm, out_hbm,
          isrc, idst, rs0, rs1, rd0, rd1, ex, dsh, sa, sb, sc2, sd2):
        cid = lax.axis_index("c")
        sid = lax.axis_index("s")
        wid = cid * 16 + sid

        def zex(e, c):
            for kk in range(8):
                ex[e, pl.ds(kk * 16, 16)] = jnp.zeros((16,), jnp.float32)
            return c

        lax.fori_loop(0, CH, zex, 0)
        pltpu.sync_copy(z_hbm, dsh.at[pl.ds(sid * STRIPE, STRIPE)])
        plsc.subcore_barrier()

        def chunk(j, carry):
            pltpu.sync_copy(src_hbm.at[wid, pl.ds(j, 1)], isrc)
            pltpu.sync_copy(dst_hbm.at[wid, pl.ds(j, 1)], idst)
            nsub = CH // CHA
            rs_b = [rs0, rs1]
            rd_b = [rd0, rd1]
            ss_b = [sa, sc2]
            sd_b = [sb, sd2]
            pend = [None] * nsub
            pend[0] = (
                pltpu.async_copy(s_hbm.at[isrc.at[0, pl.ds(0, CHA)]],
                                 rs0, sa),
                pltpu.async_copy(s_hbm.at[idst.at[0, pl.ds(0, CHA)]],
                                 rd0, sb))
            for sub in range(nsub):
                if sub + 1 < nsub:
                    nb = (sub + 1) % 2
                    pend[sub + 1] = (
                        pltpu.async_copy(
                            s_hbm.at[isrc.at[0, pl.ds((sub + 1) * CHA, CHA)]],
                            rs_b[nb], ss_b[nb]),
                        pltpu.async_copy(
                            s_hbm.at[idst.at[0, pl.ds((sub + 1) * CHA, CHA)]],
                            rd_b[nb], sd_b[nb]))
                pend[sub][0].wait()
                pend[sub][1].wait()
                rs = rs_b[sub % 2]
                rd = rd_b[sub % 2]

                def edge(e, c, _sub=sub, rs=rs, rd=rd):
                    a = rd[e, pl.ds(0, 16)] + rs[e, pl.ds(16, 16)]
                    a = jnp.maximum(a, 0.2 * a)
                    ex[_sub * CHA + e, pl.ds(0, 16)] = jnp.exp(a)
                    return c

                lax.fori_loop(0, CHA, edge, 0)
            pltpu.sync_copy(ex, dsh.at[idst.at[0]], add=True)
            return carry

        lax.fori_loop(0, W_CH, chunk, 0)
        plsc.subcore_barrier()
        pltpu.sync_copy(dsh.at[pl.ds(sid * STRIPE, STRIPE)],
                        out_hbm.at[cid, pl.ds(sid * STRIPE, STRIPE)])

    return k(s128, srcm, dstm, z128)


def _sc_agg_kernel(hhx, t2, srcm, dstm, z128):
    @functools.partial(
        pl.kernel,
        out_type=jax.ShapeDtypeStruct((2, NP, 128), jnp.float32),
        mesh=_MESH,
        scratch_types=[
            pltpu.VMEM((1, CH), jnp.int32),
            pltpu.VMEM((1, CH), jnp.int32),
            pltpu.VMEM((CHB, 128), jnp.float32),
            pltpu.VMEM((CHB, 128), jnp.float32),
            pltpu.VMEM((CHB, FDX), jnp.float32),
            pltpu.VMEM((CHB, FDX), jnp.float32),
            pltpu.VMEM((CH, 128), jnp.float32),
            pltpu.VMEM_SHARED((NP, 128), jnp.float32),
            pltpu.SemaphoreType.DMA,
            pltpu.SemaphoreType.DMA,
            pltpu.SemaphoreType.DMA,
            pltpu.SemaphoreType.DMA,
        ],
    )
    def k(hh_hbm, t2_hbm, src_hbm, dst_hbm, z_hbm, out_hbm,
          isrc, idst, rd0, rd1, rows0, rows1, msg, ash, sa, sb, sc2, sd2):
        cid = lax.axis_index("c")
        sid = lax.axis_index("s")
        wid = cid * 16 + sid

        def zmsg(e, c):
            for kk in range(8):
                msg[e, pl.ds(kk * 16, 16)] = jnp.zeros((16,), jnp.float32)
            return c

        lax.fori_loop(0, CH, zmsg, 0)
        pltpu.sync_copy(z_hbm, ash.at[pl.ds(sid * STRIPE, STRIPE)])
        plsc.subcore_barrier()

        def chunk(j, carry):
            pltpu.sync_copy(src_hbm.at[wid, pl.ds(j, 1)], isrc)
            pltpu.sync_copy(dst_hbm.at[wid, pl.ds(j, 1)], idst)
            nsub = CH // CHB
            rows_b = [rows0, rows1]
            rd_b = [rd0, rd1]
            sr_b = [sa, sc2]
            sd_b = [sb, sd2]
            pend = [None] * nsub
            pend[0] = (
                pltpu.async_copy(hh_hbm.at[isrc.at[0, pl.ds(0, CHB)]],
                                 rows0, sa),
                pltpu.async_copy(t2_hbm.at[idst.at[0, pl.ds(0, CHB)]],
                                 rd0, sb))
            for sub in range(nsub):
                if sub + 1 < nsub:
                    nb = (sub + 1) % 2
                    pend[sub + 1] = (
                        pltpu.async_copy(
                            hh_hbm.at[isrc.at[0, pl.ds((sub + 1) * CHB, CHB)]],
                            rows_b[nb], sr_b[nb]),
                        pltpu.async_copy(
                            t2_hbm.at[idst.at[0, pl.ds((sub + 1) * CHB, CHB)]],
                            rd_b[nb], sd_b[nb]))
                pend[sub][0].wait()
                pend[sub][1].wait()
                rows = rows_b[sub % 2]
                rd = rd_b[sub % 2]

                def edge(e, c, _sub=sub, rows=rows, rd=rd):
                    a = rd[e, pl.ds(0, 16)] + rows[e, pl.ds(FD, 16)]
                    a = jnp.maximum(a, 0.2 * a)
                    w = jnp.exp(a) * rd[e, pl.ds(16, 16)] * 0.125
                    acc0 = jnp.zeros((16,), jnp.float32)
                    acc1 = jnp.zeros((16,), jnp.float32)
                    acc2 = jnp.zeros((16,), jnp.float32)
                    acc3 = jnp.zeros((16,), jnp.float32)
                    for h in range(HEADS):
                        wb = lax.gather(
                            w, jnp.full((16, 1), h, jnp.int32),
                            lax.GatherDimensionNumbers(
                                offset_dims=(), collapsed_slice_dims=(0,),
                                start_index_map=(0,)),
                            (1,),
                            mode=lax.GatherScatterMode.PROMISE_IN_BOUNDS)
                        acc0 = acc0 + wb * rows[e, pl.ds(h * HID, 16)]
                        acc1 = acc1 + wb * rows[e, pl.ds(h * HID + 16, 16)]
                        acc2 = acc2 + wb * rows[e, pl.ds(h * HID + 32, 16)]
                        acc3 = acc3 + wb * rows[e, pl.ds(h * HID + 48, 16)]
                    base = _sub * CHB + e
                    msg[base, pl.ds(0, 16)] = acc0
                    msg[base, pl.ds(16, 16)] = acc1
                    msg[base, pl.ds(32, 16)] = acc2
                    msg[base, pl.ds(48, 16)] = acc3
                    return c

                lax.fori_loop(0, CHB, edge, 0)
            pltpu.sync_copy(msg, ash.at[idst.at[0]], add=True)
            return carry

        lax.fori_loop(0, W_CH, chunk, 0)
        plsc.subcore_barrier()
        pltpu.sync_copy(ash.at[pl.ds(sid * STRIPE, STRIPE)],
                        out_hbm.at[cid, pl.ds(sid * STRIPE, STRIPE)])

    return k(hhx, t2, srcm, dstm, z128)


def _make_ablk(att):
    att_i = att[0, :, :HID]
    att_j = att[0, :, HID:]
    eye = jnp.eye(HEADS, dtype=jnp.float32)
    bi = (att_i[:, :, None] * eye[:, None, :]).reshape(FD, HEADS)
    bj = (att_j[:, :, None] * eye[:, None, :]).reshape(FD, HEADS)
    return jnp.concatenate([bi, bi, bj, bj,
                            jnp.zeros((FD, 96), jnp.float32)], axis=1)


def kernel(x, edge_index, params):
    p = params
    xp = jnp.pad(x, ((0, NP - N), (0, 0)))
    srcf = jnp.concatenate(
        [edge_index[0].astype(jnp.int32), jnp.full((EP - E,), N, jnp.int32)])
    dstf = jnp.concatenate(
        [edge_index[1].astype(jnp.int32), jnp.full((EP - E,), N, jnp.int32)])
    src_m = srcf.reshape(32, W_CH, CH)
    dst_m = dstf.reshape(32, W_CH, CH)
    z128 = jnp.zeros((STRIPE, 128), jnp.float32)

    h = _encoder(xp, p)
    res = jnp.zeros((NP, HID), jnp.float32)
    for lay in p['layers']:
        ablk = _make_ablk(lay['att'])
        hhx, s128 = _fractal(h, lay, ablk)
        denp = _sc_denom_kernel(s128, src_m, dst_m, z128)
        t2 = _make_t2(denp, s128)
        accp = _sc_agg_kernel(hhx, t2, src_m, dst_m, z128)
        h, res = _post(accp, h, res, p)

    wcat = jnp.concatenate(
        [p['dec_state_W'], p['dec_importance_W'], p['dec_chaos_W'],
         p['dec_evolution_W'],
         jnp.zeros((HID, 128 - 3 - HID), jnp.float32)], axis=1)
    bcat = jnp.concatenate(
        [p['dec_state_b'], p['dec_importance_b'], p['dec_chaos_b'],
         p['dec_evolution_b'],
         jnp.zeros((128 - 3 - HID,), jnp.float32)]).reshape(1, 128)
    out128, hm = _decode(res, wcat, bcat)
    return (out128[:N, 0:1], out128[:N, 1:2], out128[:N, 2:3],
            out128[:N, 3:3 + HID], hm[:N])


# speedup vs baseline: 18.5845x; 1.0342x over previous
import functools

import jax
import jax.numpy as jnp
from jax import lax
from jax.experimental import pallas as pl
from jax.experimental.pallas import tpu as pltpu
from jax.experimental.pallas import tpu_sc as plsc

N = 10000
NP = 10240
E = 160000
EP = 163840
D_FEAT = 128
HID = 64
HEADS = 8
FD = 512
FDX = 640
ROWS = 320
GRID = NP // ROWS
W_CH = 40
CH = 128
CHA = 32
CHB = 16
STRIPE = NP // 16


def _ln(x, g, b):
    mu = jnp.mean(x, axis=-1, keepdims=True)
    var = jnp.mean((x - mu) ** 2, axis=-1, keepdims=True)
    return (x - mu) * jax.lax.rsqrt(var + 1e-5) * g + b


def _ln_plain(x):
    mu = jnp.mean(x, axis=-1, keepdims=True)
    var = jnp.mean((x - mu) ** 2, axis=-1, keepdims=True)
    return (x - mu) * jax.lax.rsqrt(var + 1e-5)


def _enc_body(x_ref, w_ref, b_ref, g_ref, be_ref, o_ref):
    h = jnp.dot(x_ref[...], w_ref[...], preferred_element_type=jnp.float32)
    o_ref[...] = jax.nn.relu(_ln(h + b_ref[...], g_ref[...], be_ref[...]))


def _encoder(xp, p):
    return pl.pallas_call(
        _enc_body,
        grid=(GRID,),
        in_specs=[
            pl.BlockSpec((ROWS, D_FEAT), lambda i: (i, 0)),
            pl.BlockSpec((D_FEAT, HID), lambda i: (0, 0)),
            pl.BlockSpec((1, HID), lambda i: (0, 0)),
            pl.BlockSpec((1, HID), lambda i: (0, 0)),
            pl.BlockSpec((1, HID), lambda i: (0, 0)),
        ],
        out_specs=pl.BlockSpec((ROWS, HID), lambda i: (i, 0)),
        out_shape=jax.ShapeDtypeStruct((NP, HID), jnp.float32),
    )(xp, p['enc_W'], p['enc_b'].reshape(1, HID), p['enc_g'].reshape(1, HID),
      p['enc_be'].reshape(1, HID))


def _frac_body(h_ref, w0, b0, g0, e0, w1, b1, g1, e1, w2, b2, g2, e2, ab,
               hhx_ref, s_ref):
    t0 = jax.nn.relu(_ln(jnp.dot(h_ref[...], w0[...],
                                 preferred_element_type=jnp.float32) + b0[...],
                         g0[...], e0[...]))
    t1 = jax.nn.relu(_ln(jnp.dot(t0, w1[...],
                                 preferred_element_type=jnp.float32) + b1[...],
                         g1[...], e1[...]))
    t2 = jax.nn.relu(_ln(jnp.dot(t1, w2[...],
                                 preferred_element_type=jnp.float32) + b2[...],
                         g2[...], e2[...]))
    hh = (t0 + t1 + t2) * (1.0 / 3.0)
    s = jnp.dot(hh, ab[...], preferred_element_type=jnp.float32)
    s_ref[...] = s
    hhx_ref[...] = jnp.concatenate(
        [hh, s[:, 16:32], jnp.zeros((hh.shape[0], FDX - FD - 16),
                                    jnp.float32)], axis=1)


def _fractal(h, lay, ablk):
    t = lay['t']
    full = lambda shape: pl.BlockSpec(shape, lambda i: (0, 0))
    args = [h]
    specs = [pl.BlockSpec((ROWS, HID), lambda i: (i, 0))]
    dims = [HID, FD, FD]
    for d in range(3):
        args += [t[d]['W'], t[d]['b'].reshape(1, FD), t[d]['g'].reshape(1, FD),
                 t[d]['be'].reshape(1, FD)]
        specs += [full((dims[d], FD)), full((1, FD)), full((1, FD)),
                  full((1, FD))]
    args.append(ablk)
    specs.append(full((FD, 128)))
    return pl.pallas_call(
        _frac_body,
        grid=(GRID,),
        in_specs=specs,
        out_specs=[pl.BlockSpec((ROWS, FDX), lambda i: (i, 0)),
                   pl.BlockSpec((ROWS, 128), lambda i: (i, 0))],
        out_shape=[jax.ShapeDtypeStruct((NP, FDX), jnp.float32),
                   jax.ShapeDtypeStruct((NP, 128), jnp.float32)],
    )(*args)


def _t2_body(dp_ref, s_ref, o_ref):
    dinv = 1.0 / (dp_ref[0, :, 0:16] + dp_ref[1, :, 0:16] + 1e-16)
    o_ref[...] = jnp.concatenate(
        [s_ref[:, 0:16], dinv,
         jnp.zeros((dinv.shape[0], 96), jnp.float32)], axis=1)


def _make_t2(denp, s):
    return pl.pallas_call(
        _t2_body,
        grid=(GRID,),
        in_specs=[pl.BlockSpec((2, ROWS, 128), lambda i: (0, i, 0)),
                  pl.BlockSpec((ROWS, 128), lambda i: (i, 0))],
        out_specs=pl.BlockSpec((ROWS, 128), lambda i: (i, 0)),
        out_shape=jax.ShapeDtypeStruct((NP, 128), jnp.float32),
    )(denp, s)


def _post_body(a_ref, hp_ref, r_ref, w1, b1, w2, b2, hl_ref, ro_ref):
    agg = a_ref[0, :, 0:HID] + a_ref[1, :, 0:HID]
    e1 = jax.nn.relu(jnp.dot(agg, w1[...],
                             preferred_element_type=jnp.float32) + b1[...])
    evo = jnp.tanh(jnp.dot(e1, w2[...],
                           preferred_element_type=jnp.float32) + b2[...])
    gate = jax.nn.sigmoid(evo)
    hn = gate * agg + (1.0 - gate) * hp_ref[...]
    ro_ref[...] = r_ref[...] + hn
    hl_ref[...] = _ln_plain(hn)


def _post(accp, h_prev, res, p):
    full = lambda shape: pl.BlockSpec(shape, lambda i: tuple(0 for _ in shape))
    return pl.pallas_call(
        _post_body,
        grid=(GRID,),
        in_specs=[
            pl.BlockSpec((2, ROWS, 128), lambda i: (0, i, 0)),
            pl.BlockSpec((ROWS, HID), lambda i: (i, 0)),
            pl.BlockSpec((ROWS, HID), lambda i: (i, 0)),
            full((HID, 2 * HID)), full((1, 2 * HID)),
            full((2 * HID, HID)), full((1, HID)),
        ],
        out_specs=[pl.BlockSpec((ROWS, HID), lambda i: (i, 0)),
                   pl.BlockSpec((ROWS, HID), lambda i: (i, 0))],
        out_shape=[jax.ShapeDtypeStruct((NP, HID), jnp.float32),
                   jax.ShapeDtypeStruct((NP, HID), jnp.float32)],
    )(accp, h_prev, res, p['evo_W1'], p['evo_b1'].reshape(1, 2 * HID),
      p['evo_W2'], p['evo_b2'].reshape(1, HID))


def _dec_body(r_ref, w_ref, b_ref, o_ref, hm_ref):
    hm = r_ref[...] * (1.0 / 3.0)
    hm_ref[...] = hm
    o_ref[...] = jnp.dot(hm, w_ref[...],
                         preferred_element_type=jnp.float32) + b_ref[...]


def _decode(res, wcat, bcat):
    return pl.pallas_call(
        _dec_body,
        grid=(GRID,),
        in_specs=[
            pl.BlockSpec((ROWS, HID), lambda i: (i, 0)),
            pl.BlockSpec((HID, 128), lambda i: (0, 0)),
            pl.BlockSpec((1, 128), lambda i: (0, 0)),
        ],
        out_specs=[pl.BlockSpec((ROWS, 128), lambda i: (i, 0)),
                   pl.BlockSpec((ROWS, HID), lambda i: (i, 0))],
        out_shape=[jax.ShapeDtypeStruct((NP, 128), jnp.float32),
                   jax.ShapeDtypeStruct((NP, HID), jnp.float32)],
    )(res, wcat, bcat)


_MESH = plsc.VectorSubcoreMesh(core_axis_name="c", subcore_axis_name="s")


def _sc_denom_kernel(s128, srcm, dstm, z128):
    @functools.partial(
        pl.kernel,
        out_type=jax.ShapeDtypeStruct((2, NP, 128), jnp.float32),
        mesh=_MESH,
        scratch_types=[
            pltpu.VMEM((W_CH, CH), jnp.int32),
            pltpu.VMEM((W_CH, CH), jnp.int32),
            pltpu.VMEM((CHA, 128), jnp.float32),
            pltpu.VMEM((CHA, 128), jnp.float32),
            pltpu.VMEM((CHA, 128), jnp.float32),
            pltpu.VMEM((CHA, 128), jnp.float32),
            pltpu.VMEM((CH, 128), jnp.float32),
            pltpu.VMEM_SHARED((NP, 128), jnp.float32),
            pltpu.SemaphoreType.DMA,
            pltpu.SemaphoreType.DMA,
            pltpu.SemaphoreType.DMA,
            pltpu.SemaphoreType.DMA,
        ],
    )
    def k(s_hbm, src_hbm, dst_hbm, z_hbm, out_hbm,
          isrc, idst, rs0, rs1, rd0, rd1, ex, dsh, sa, sb, sc2, sd2):

        cid = lax.axis_index("c")
        sid = lax.axis_index("s")
        wid = cid * 16 + sid

        def zex(e, c):
            for kk in range(8):
                ex[e, pl.ds(kk * 16, 16)] = jnp.zeros((16,), jnp.float32)
            return c

        lax.fori_loop(0, CH, zex, 0)
        pltpu.sync_copy(z_hbm, dsh.at[pl.ds(sid * STRIPE, STRIPE)])
        pltpu.sync_copy(src_hbm.at[wid], isrc)
        pltpu.sync_copy(dst_hbm.at[wid], idst)
        plsc.subcore_barrier()

        def chunk(j, carry):
            nsub = CH // CHA
            rs_b = [rs0, rs1]
            rd_b = [rd0, rd1]
            ss_b = [sa, sc2]
            sd_b = [sb, sd2]
            pend = [None] * nsub
            pend[0] = (
                pltpu.async_copy(s_hbm.at[isrc.at[j, pl.ds(0, CHA)]],
                                 rs0, sa),
                pltpu.async_copy(s_hbm.at[idst.at[j, pl.ds(0, CHA)]],
                                 rd0, sb))
            for sub in range(nsub):
                if sub + 1 < nsub:
                    nb = (sub + 1) % 2
                    pend[sub + 1] = (
                        pltpu.async_copy(
                            s_hbm.at[isrc.at[j, pl.ds((sub + 1) * CHA, CHA)]],
                            rs_b[nb], ss_b[nb]),
                        pltpu.async_copy(
                            s_hbm.at[idst.at[j, pl.ds((sub + 1) * CHA, CHA)]],
                            rd_b[nb], sd_b[nb]))
                pend[sub][0].wait()
                pend[sub][1].wait()
                rs = rs_b[sub % 2]
                rd = rd_b[sub % 2]

                def edge(e, c, _sub=sub, rs=rs, rd=rd):
                    a = rd[e, pl.ds(0, 16)] + rs[e, pl.ds(16, 16)]
                    a = jnp.maximum(a, 0.2 * a)
                    ex[_sub * CHA + e, pl.ds(0, 16)] = jnp.exp(a)
                    return c

                lax.fori_loop(0, CHA, edge, 0)
            pltpu.sync_copy(ex, dsh.at[idst.at[j]], add=True)
            return carry

        lax.fori_loop(0, W_CH, chunk, 0)
        plsc.subcore_barrier()
        pltpu.sync_copy(dsh.at[pl.ds(sid * STRIPE, STRIPE)],
                        out_hbm.at[cid, pl.ds(sid * STRIPE, STRIPE)])

    return k(s128, srcm, dstm, z128)


def _sc_agg_kernel(hhx, t2, srcm, dstm, z128):
    @functools.partial(
        pl.kernel,
        out_type=jax.ShapeDtypeStruct((2, NP, 128), jnp.float32),
        mesh=_MESH,
        scratch_types=[
            pltpu.VMEM((W_CH, CH), jnp.int32),
            pltpu.VMEM((1, CH), jnp.int32),
            pltpu.VMEM((CHB, 128), jnp.float32),
            pltpu.VMEM((CHB, 128), jnp.float32),
            pltpu.VMEM((CHB, FDX), jnp.float32),
            pltpu.VMEM((CHB, FDX), jnp.float32),
            pltpu.VMEM((CH, 128), jnp.float32),
            pltpu.VMEM_SHARED((NP, 128), jnp.float32),
            pltpu.SemaphoreType.DMA,
            pltpu.SemaphoreType.DMA,
            pltpu.SemaphoreType.DMA,
            pltpu.SemaphoreType.DMA,
        ],
    )
    def k(hh_hbm, t2_hbm, src_hbm, dst_hbm, z_hbm, out_hbm,
          isrc, idst, rd0, rd1, rows0, rows1, msg, ash, sa, sb, sc2, sd2):
        cid = lax.axis_index("c")
        sid = lax.axis_index("s")
        wid = cid * 16 + sid

        def zmsg(e, c):
            for kk in range(8):
                msg[e, pl.ds(kk * 16, 16)] = jnp.zeros((16,), jnp.float32)
            return c

        lax.fori_loop(0, CH, zmsg, 0)
        pltpu.sync_copy(z_hbm, ash.at[pl.ds(sid * STRIPE, STRIPE)])
        pltpu.sync_copy(src_hbm.at[wid], isrc)
        plsc.subcore_barrier()

        def chunk(j, carry):
            pltpu.sync_copy(dst_hbm.at[wid, pl.ds(j, 1)], idst)
            nsub = CH // CHB
            rows_b = [rows0, rows1]
            rd_b = [rd0, rd1]
            sr_b = [sa, sc2]
            sd_b = [sb, sd2]
            pend = [None] * nsub
            pend[0] = (
                pltpu.async_copy(hh_hbm.at[isrc.at[j, pl.ds(0, CHB)]],
                                 rows0, sa),
                pltpu.async_copy(t2_hbm.at[idst.at[0, pl.ds(0, CHB)]],
                                 rd0, sb))
            for sub in range(nsub):
                if sub + 1 < nsub:
                    nb = (sub + 1) % 2
                    pend[sub + 1] = (
                        pltpu.async_copy(
                            hh_hbm.at[isrc.at[j, pl.ds((sub + 1) * CHB, CHB)]],
                            rows_b[nb], sr_b[nb]),
                        pltpu.async_copy(
                            t2_hbm.at[idst.at[0, pl.ds((sub + 1) * CHB, CHB)]],
                            rd_b[nb], sd_b[nb]))
                pend[sub][0].wait()
                pend[sub][1].wait()
                rows = rows_b[sub % 2]
                rd = rd_b[sub % 2]

                def edge(e, c, _sub=sub, rows=rows, rd=rd):
                    a = rd[e, pl.ds(0, 16)] + rows[e, pl.ds(FD, 16)]
                    a = jnp.maximum(a, 0.2 * a)
                    w = jnp.exp(a) * rd[e, pl.ds(16, 16)] * 0.125
                    acc0 = jnp.zeros((16,), jnp.float32)
                    acc1 = jnp.zeros((16,), jnp.float32)
                    acc2 = jnp.zeros((16,), jnp.float32)
                    acc3 = jnp.zeros((16,), jnp.float32)
                    for h in range(HEADS):
                        wb = lax.gather(
                            w, jnp.full((16, 1), h, jnp.int32),
                            lax.GatherDimensionNumbers(
                                offset_dims=(), collapsed_slice_dims=(0,),
                                start_index_map=(0,)),
                            (1,),
                            mode=lax.GatherScatterMode.PROMISE_IN_BOUNDS)
                        acc0 = acc0 + wb * rows[e, pl.ds(h * HID, 16)]
                        acc1 = acc1 + wb * rows[e, pl.ds(h * HID + 16, 16)]
                        acc2 = acc2 + wb * rows[e, pl.ds(h * HID + 32, 16)]
                        acc3 = acc3 + wb * rows[e, pl.ds(h * HID + 48, 16)]
                    base = _sub * CHB + e
                    msg[base, pl.ds(0, 16)] = acc0
                    msg[base, pl.ds(16, 16)] = acc1
                    msg[base, pl.ds(32, 16)] = acc2
                    msg[base, pl.ds(48, 16)] = acc3
                    return c

                lax.fori_loop(0, CHB, edge, 0)
            pltpu.sync_copy(msg, ash.at[idst.at[0]], add=True)
            return carry

        lax.fori_loop(0, W_CH, chunk, 0)
        plsc.subcore_barrier()
        pltpu.sync_copy(ash.at[pl.ds(sid * STRIPE, STRIPE)],
                        out_hbm.at[cid, pl.ds(sid * STRIPE, STRIPE)])

    return k(hhx, t2, srcm, dstm, z128)


def _make_ablk(att):
    att_i = att[0, :, :HID]
    att_j = att[0, :, HID:]
    eye = jnp.eye(HEADS, dtype=jnp.float32)
    bi = (att_i[:, :, None] * eye[:, None, :]).reshape(FD, HEADS)
    bj = (att_j[:, :, None] * eye[:, None, :]).reshape(FD, HEADS)
    return jnp.concatenate([bi, bi, bj, bj,
                            jnp.zeros((FD, 96), jnp.float32)], axis=1)


def kernel(x, edge_index, params):
    p = params
    xp = jnp.pad(x, ((0, NP - N), (0, 0)))
    srcf = jnp.concatenate(
        [edge_index[0].astype(jnp.int32), jnp.full((EP - E,), N, jnp.int32)])
    dstf = jnp.concatenate(
        [edge_index[1].astype(jnp.int32), jnp.full((EP - E,), N, jnp.int32)])
    src_m = srcf.reshape(32, W_CH, CH)
    dst_m = dstf.reshape(32, W_CH, CH)
    z128 = jnp.zeros((STRIPE, 128), jnp.float32)

    h = _encoder(xp, p)
    res = jnp.zeros((NP, HID), jnp.float32)
    for lay in p['layers']:
        ablk = _make_ablk(lay['att'])
        hhx, s128 = _fractal(h, lay, ablk)
        denp = _sc_denom_kernel(s128, src_m, dst_m, z128)
        t2 = _make_t2(denp, s128)
        accp = _sc_agg_kernel(hhx, t2, src_m, dst_m, z128)
        h, res = _post(accp, h, res, p)

    wcat = jnp.concatenate(
        [p['dec_state_W'], p['dec_importance_W'], p['dec_chaos_W'],
         p['dec_evolution_W'],
         jnp.zeros((HID, 128 - 3 - HID), jnp.float32)], axis=1)
    bcat = jnp.concatenate(
        [p['dec_state_b'], p['dec_importance_b'], p['dec_chaos_b'],
         p['dec_evolution_b'],
         jnp.zeros((128 - 3 - HID,), jnp.float32)]).reshape(1, 128)
    out128, hm = _decode(res, wcat, bcat)
    return (out128[:N, 0:1], out128[:N, 1:2], out128[:N, 2:3],
            out128[:N, 3:3 + HID], hm[:N])
